# baseline (speedup 1.0000x reference)
import functools

import jax
import jax.numpy as jnp
from jax import lax
from jax.experimental import pallas as pl
from jax.experimental.pallas import tpu as pltpu
from jax.experimental.pallas import tpu_sc as plsc

N = 320000
D = 128
S = 10000
S_PAD = 10496
EPS = 1e-5

NC = 2
NS = 16
NW = NC * NS
ROWS_W = N // NW
SUB = 80
NSUB = 5
CHUNK = SUB * NSUB
NCHUNK = ROWS_W // CHUNK

R = 4000
NB = N // R
W = 256


def _sc_gather(h_hbm, ids_hbm, g_hbm, idx_buf, rows_buf, sem):
    c = lax.axis_index("c")
    s = lax.axis_index("s")
    wid = c * NS + s

    def chunk_body(k, carry):
        pltpu.sync_copy(ids_hbm.at[wid, k], idx_buf)
        copies = [
            pltpu.async_copy(h_hbm.at[idx_buf.at[j]],
                             rows_buf.at[pl.ds(j * SUB, SUB), :], sem)
            for j in range(NSUB)
        ]
        for cp in copies:
            cp.wait()
        row0 = wid * ROWS_W + k * CHUNK
        pltpu.sync_copy(rows_buf, g_hbm.at[pl.ds(row0, CHUNK), :])
        return carry

    lax.fori_loop(0, NCHUNK, chunk_body, 0)


def _sc_gather_call(h, ids4d):
    fn = functools.partial(
        pl.kernel,
        out_type=jax.ShapeDtypeStruct((N, D), jnp.float32),
        mesh=plsc.VectorSubcoreMesh(core_axis_name="c", subcore_axis_name="s",
                                    num_cores=NC, num_subcores=NS),
        scratch_types=[
            pltpu.VMEM((NSUB, SUB), jnp.int32),
            pltpu.VMEM((CHUNK, D), jnp.float32),
            pltpu.SemaphoreType.DMA,
        ],
    )(_sc_gather)
    return fn(h, ids4d)


def _seg_mlp_kernel(ids_smem_ref, idsr_ref, x_ref, w1_ref, b1_ref, w2_ref,
                    b2_ref, h_ref, acc_ref, cacc_ref):
    i = pl.program_id(0)

    @pl.when(i == 0)
    def _init():
        acc_ref[...] = jnp.zeros((S_PAD, D), jnp.float32)
        cacc_ref[...] = jnp.zeros((S_PAD, D), jnp.float32)

    @pl.when(i < NB)
    def _accum():
        base = ids_smem_ref[0, 0, 0]
        last = ids_smem_ref[0, 0, R - 1]
        off0 = (base // 8) * 8
        nwin = (last - off0) // W + 1

        idsr = idsr_ref[0]
        iota = lax.broadcasted_iota(jnp.int32, (W, R), 0)
        xbf = x_ref[...].astype(jnp.bfloat16)
        ones = jnp.ones((R, D), jnp.bfloat16)
        dims = (((1,), (0,)), ((), ()))

        def accum_win(off):
            oht = (iota == idsr - off).astype(jnp.bfloat16)
            st = jax.lax.dot_general(oht, xbf, dims,
                                     preferred_element_type=jnp.float32)
            ct = jax.lax.dot_general(oht, ones, dims,
                                     preferred_element_type=jnp.float32)
            acc_ref[pl.ds(off, W), :] += st
            cacc_ref[pl.ds(off, W), :] += ct

        accum_win(off0)

        def win_body(k, carry):
            accum_win(off0 + k * W)
            return carry

        @pl.when(nwin > 1)
        def _extra():
            lax.fori_loop(1, nwin, win_body, 0)

    @pl.when(i == NB)
    def _mlp():
        mean = acc_ref[...] / jnp.maximum(cacc_ref[...], 1.0)
        h = jnp.dot(mean, w1_ref[...], preferred_element_type=jnp.float32)
        h = jnp.maximum(h + b1_ref[...], 0.0)
        h = jnp.dot(h, w2_ref[...], preferred_element_type=jnp.float32)
        h_ref[...] = h + b2_ref[...]


def _pass2_kernel(x_ref, g_ref, gamma_ref, beta_ref, out_ref):
    o = x_ref[...] + g_ref[...]
    dims = (((1,), (0,)), ((), ()))
    jd = jnp.full((D, D), 1.0 / D, dtype=jnp.bfloat16)
    mu = jax.lax.dot_general(o.astype(jnp.bfloat16), jd, dims,
                             preferred_element_type=jnp.float32)
    d = o - mu
    msq = jax.lax.dot_general((d * d).astype(jnp.bfloat16), jd, dims,
                              preferred_element_type=jnp.float32)
    rstd = lax.rsqrt(msq + EPS)
    out_ref[...] = d * rstd * gamma_ref[...] + beta_ref[...]


def kernel(x, subnet_id, W1, b1, W2, b2, gamma, beta):
    ids = subnet_id.astype(jnp.int32)
    ids4d = ids.reshape(NW, NCHUNK, NSUB, SUB)
    ids_blk = ids.reshape(NB, 1, R)

    clamp = lambda i: (jnp.minimum(i, NB - 1), 0, 0)
    h = pl.pallas_call(
        _seg_mlp_kernel,
        grid=(NB + 1,),
        in_specs=[
            pl.BlockSpec((1, 1, R), clamp, memory_space=pltpu.SMEM),
            pl.BlockSpec((1, 1, R), clamp),
            pl.BlockSpec((R, D), lambda i: (jnp.minimum(i, NB - 1), 0)),
            pl.BlockSpec((D, D), lambda i: (0, 0)),
            pl.BlockSpec((1, D), lambda i: (0, 0)),
            pl.BlockSpec((D, D), lambda i: (0, 0)),
            pl.BlockSpec((1, D), lambda i: (0, 0)),
        ],
        out_specs=pl.BlockSpec((S_PAD, D), lambda i: (0, 0)),
        out_shape=jax.ShapeDtypeStruct((S_PAD, D), jnp.float32),
        scratch_shapes=[
            pltpu.VMEM((S_PAD, D), jnp.float32),
            pltpu.VMEM((S_PAD, D), jnp.float32),
        ],
    )(ids_blk, ids_blk, x, W1, b1.reshape(1, D), W2, b2.reshape(1, D))

    g = _sc_gather_call(h, ids4d)

    out = pl.pallas_call(
        _pass2_kernel,
        grid=(NB,),
        in_specs=[
            pl.BlockSpec((R, D), lambda i: (i, 0)),
            pl.BlockSpec((R, D), lambda i: (i, 0)),
            pl.BlockSpec((1, D), lambda i: (0, 0)),
            pl.BlockSpec((1, D), lambda i: (0, 0)),
        ],
        out_specs=pl.BlockSpec((R, D), lambda i: (i, 0)),
        out_shape=jax.ShapeDtypeStruct((N, D), jnp.float32),
    )(x, g, gamma.reshape(1, D), beta.reshape(1, D))
    return out
